# baseline (speedup 1.0000x reference)
import functools

import jax
import jax.numpy as jnp
from jax import lax
from jax.experimental import pallas as pl
from jax.experimental.pallas import tpu as pltpu
from jax.experimental.pallas import tpu_sc as plsc

B, D = 16384, 64
NC, NS, L = 2, 16, 16
NW = NC * NS
RPW = B // NW
CH = 128
RING = 64
NG = RPW // L
LEAD = RING // L


def _sc_center_loss(featsT, label2d, centers):
    mesh = plsc.VectorSubcoreMesh(core_axis_name="c", subcore_axis_name="s")

    @functools.partial(
        pl.kernel,
        mesh=mesh,
        compiler_params=pltpu.CompilerParams(use_tc_tiling_on_sc=True),
        out_type=jax.ShapeDtypeStruct((NW, L), jnp.float32),
        scratch_types=[
            pltpu.VMEM((RPW // CH, CH), jnp.int32),
            pltpu.VMEM_SHARED((NS, D, RPW), jnp.float32),
            pltpu.VMEM((RPW, D + 1), jnp.float32),
            pltpu.VMEM((RING, D), jnp.float32),
            pltpu.VMEM((L,), jnp.float32),
            pltpu.SemaphoreType.DMA,
        ],
    )
    def k(featsT_hbm, label_hbm, centers_hbm, out_hbm,
          lab_v, ft_v, f_v, c_ring, acc_v, sem):
        wid = lax.axis_index("s") * NC + lax.axis_index("c")

        pltpu.sync_copy(label_hbm.at[pl.ds(wid * (RPW // CH), RPW // CH)],
                        lab_v)

        def labs_of(g):
            return lab_v[g // (CH // L), pl.ds((g % (CH // L)) * L, L)]

        def issue_group(g, labs):
            for u in range(L):
                r = g * L + u
                pltpu.async_copy(centers_hbm.at[labs[u]],
                                 c_ring.at[r % RING], sem)

        for g in range(LEAD):
            issue_group(g, labs_of(g))
        sid = lax.axis_index("s")
        pltpu.sync_copy(featsT_hbm.at[:, pl.ds(wid * RPW, RPW)], ft_v.at[sid])
        for d in range(D):
            pltpu.sync_copy(ft_v.at[sid, d], f_v.at[:, d])

        lane = lax.iota(jnp.int32, 16)
        is_last = lane == 15
        zero = jnp.zeros((L,), jnp.float32)
        perms = [lane ^ sh for sh in (8, 4, 2, 1)]

        def body(g, acc):
            labs_next = labs_of(jnp.minimum(g + LEAD, NG - 1))
            for q in range(L // 4):
                r0 = g * L + q * 4
                pltpu.make_async_copy(
                    centers_hbm.at[pl.ds(0, 4)],
                    c_ring.at[pl.ds(r0 % RING, 4)], sem).wait()
                for u in range(4):
                    r = r0 + u
                    s = None
                    for cc in range(D // L):
                        df = (f_v[r, pl.ds(cc * L, L)]
                              - c_ring[r % RING, pl.ds(cc * L, L)])
                        sq = df * df
                        s = sq if s is None else s + sq

                    @pl.when(g < NG - LEAD)
                    def _():
                        pltpu.async_copy(centers_hbm.at[labs_next[q * 4 + u]],
                                         c_ring.at[r % RING], sem)

                    for p in perms:
                        s = s + s.at[p].get(mode="promise_in_bounds")
                    w = jnp.clip(s * 0.5, 1e-12, 1e12)
                    acc = acc + jnp.where(is_last, w, zero)
            return acc

        acc = lax.fori_loop(0, NG, body, zero)
        acc_v[...] = acc
        pltpu.sync_copy(acc_v, out_hbm.at[wid])

    return k(featsT, label2d, centers)


def kernel(feats, label, centers):
    label2d = label.reshape(B // CH, CH)
    partials = _sc_center_loss(feats.T, label2d, centers)
    return jnp.sum(partials) / 16384.0


# speedup vs baseline: 2.5822x; 2.5822x over previous
import functools

import jax
import jax.numpy as jnp
from jax import lax
from jax.experimental import pallas as pl
from jax.experimental.pallas import tpu as pltpu
from jax.experimental.pallas import tpu_sc as plsc

B, D = 16384, 64
NC, NS, L = 2, 16, 16
NW = NC * NS
RPW = B // NW
CH = 128
RING = 64
NG = RPW // L
LEAD = RING // L


def _sc_center_loss(feats, label2d, centers):
    mesh = plsc.VectorSubcoreMesh(core_axis_name="c", subcore_axis_name="s")

    @functools.partial(
        pl.kernel,
        mesh=mesh,
        compiler_params=pltpu.CompilerParams(use_tc_tiling_on_sc=True),
        out_type=jax.ShapeDtypeStruct((NW, L), jnp.float32),
        scratch_types=[
            pltpu.VMEM((RPW // CH, CH), jnp.int32),
            pltpu.VMEM((RPW, D), jnp.float32),
            pltpu.VMEM((RING, D), jnp.float32),
            pltpu.VMEM((L,), jnp.float32),
            pltpu.SemaphoreType.DMA,
        ],
    )
    def k(feats_hbm, label_hbm, centers_hbm, out_hbm,
          lab_v, f_v, c_ring, acc_v, sem):
        wid = lax.axis_index("s") * NC + lax.axis_index("c")

        pltpu.sync_copy(label_hbm.at[pl.ds(wid * (RPW // CH), RPW // CH)],
                        lab_v)

        def labs_of(g):
            return lab_v[g // (CH // L), pl.ds((g % (CH // L)) * L, L)]

        def issue_group(g, labs):
            for u in range(L):
                r = g * L + u
                pltpu.async_copy(centers_hbm.at[labs[u]],
                                 c_ring.at[r % RING], sem)

        for g in range(LEAD):
            issue_group(g, labs_of(g))
        pltpu.sync_copy(feats_hbm.at[pl.ds(wid * RPW, RPW)], f_v)

        lane = lax.iota(jnp.int32, 16)
        is_last = lane == 15
        zero = jnp.zeros((L,), jnp.float32)
        perms = [lane ^ sh for sh in (8, 4, 2, 1)]

        def body(g, acc):
            labs_next = labs_of(jnp.minimum(g + LEAD, NG - 1))
            for q in range(L // 4):
                r0 = g * L + q * 4
                pltpu.make_async_copy(
                    centers_hbm.at[pl.ds(0, 4)],
                    c_ring.at[pl.ds(r0 % RING, 4)], sem).wait()
                for u in range(4):
                    r = r0 + u
                    s = None
                    for cc in range(D // L):
                        df = (f_v[r, pl.ds(cc * L, L)]
                              - c_ring[r % RING, pl.ds(cc * L, L)])
                        sq = df * df
                        s = sq if s is None else s + sq

                    @pl.when(g < NG - LEAD)
                    def _():
                        pltpu.async_copy(centers_hbm.at[labs_next[q * 4 + u]],
                                         c_ring.at[r % RING], sem)

                    for p in perms:
                        s = s + s.at[p].get(mode="promise_in_bounds")
                    w = jnp.clip(s * 0.5, 1e-12, 1e12)
                    acc = acc + jnp.where(is_last, w, zero)
            return acc

        acc = lax.fori_loop(0, NG, body, zero)
        acc_v[...] = acc
        pltpu.sync_copy(acc_v, out_hbm.at[wid])

    return k(feats, label2d, centers)


def kernel(feats, label, centers):
    label2d = label.reshape(B // CH, CH)
    partials = _sc_center_loss(feats, label2d, centers)
    return jnp.sum(partials) / 16384.0


# speedup vs baseline: 2.6144x; 1.0125x over previous
import functools

import jax
import jax.numpy as jnp
from jax import lax
from jax.experimental import pallas as pl
from jax.experimental.pallas import tpu as pltpu
from jax.experimental.pallas import tpu_sc as plsc

B, D = 16384, 64
NC, NS, L = 2, 16, 16
NW = NC * NS
RPW = B // NW
CH = 128
RING = 128
NG = RPW // L
LEAD = RING // L


def _sc_center_loss(feats, label2d, centers):
    mesh = plsc.VectorSubcoreMesh(core_axis_name="c", subcore_axis_name="s")

    @functools.partial(
        pl.kernel,
        mesh=mesh,
        compiler_params=pltpu.CompilerParams(use_tc_tiling_on_sc=True),
        out_type=jax.ShapeDtypeStruct((NW, L), jnp.float32),
        scratch_types=[
            pltpu.VMEM((RPW // CH, CH), jnp.int32),
            pltpu.VMEM((RPW, D), jnp.float32),
            pltpu.VMEM((RING, D), jnp.float32),
            pltpu.VMEM((L,), jnp.float32),
            pltpu.SemaphoreType.DMA,
        ],
    )
    def k(feats_hbm, label_hbm, centers_hbm, out_hbm,
          lab_v, f_v, c_ring, acc_v, sem):
        wid = lax.axis_index("s") * NC + lax.axis_index("c")

        pltpu.sync_copy(label_hbm.at[pl.ds(wid * (RPW // CH), RPW // CH)],
                        lab_v)

        def labs_of(g):
            return lab_v[g // (CH // L), pl.ds((g % (CH // L)) * L, L)]

        def issue_group(g, labs):
            for u in range(L):
                r = g * L + u
                pltpu.async_copy(centers_hbm.at[labs[u]],
                                 c_ring.at[r % RING], sem)

        for g in range(LEAD):
            issue_group(g, labs_of(g))
        pltpu.sync_copy(feats_hbm.at[pl.ds(wid * RPW, RPW)], f_v)

        lane = lax.iota(jnp.int32, 16)
        is_last = lane == 15
        zero = jnp.zeros((L,), jnp.float32)
        perms = [lane ^ sh for sh in (8, 4, 2, 1)]

        def body(g, acc):
            labs_next = labs_of(jnp.minimum(g + LEAD, NG - 1))
            for q in range(L // 4):
                r0 = g * L + q * 4
                pltpu.make_async_copy(
                    centers_hbm.at[pl.ds(0, 4)],
                    c_ring.at[pl.ds(r0 % RING, 4)], sem).wait()
                for u in range(4):
                    r = r0 + u
                    s = None
                    for cc in range(D // L):
                        df = (f_v[r, pl.ds(cc * L, L)]
                              - c_ring[r % RING, pl.ds(cc * L, L)])
                        sq = df * df
                        s = sq if s is None else s + sq

                    @pl.when(g < NG - LEAD)
                    def _():
                        pltpu.async_copy(centers_hbm.at[labs_next[q * 4 + u]],
                                         c_ring.at[r % RING], sem)

                    for p in perms:
                        s = s + s.at[p].get(mode="promise_in_bounds")
                    w = jnp.clip(s * 0.5, 1e-12, 1e12)
                    acc = acc + jnp.where(is_last, w, zero)
            return acc

        acc = lax.fori_loop(0, NG, body, zero)
        acc_v[...] = acc
        pltpu.sync_copy(acc_v, out_hbm.at[wid])

    return k(feats, label2d, centers)


def kernel(feats, label, centers):
    label2d = label.reshape(B // CH, CH)
    partials = _sc_center_loss(feats, label2d, centers)
    return jnp.sum(partials) / 16384.0
